# baseline (speedup 1.0000x reference)
import functools

import jax
import jax.numpy as jnp
from jax import lax
from jax.experimental import pallas as pl
from jax.experimental.pallas import tpu as pltpu
from jax.experimental.pallas import tpu_sc as plsc

N_TOK = 4096
D_IN = 768
NUM_EXPERTS = 8
NUM_SPIKE = 2
TOTAL = NUM_EXPERTS + NUM_SPIKE
EXPERT_DIM = 768
SPIKE_CAP = 1536
OUT_DIM = 768
SPIKE_LEN = 16

BT = 256
NBLK_REG_MAX = 40
NBLK_SPK_MAX = 34
NBLK = 43
A_CAP = NBLK * BT
GRID_E = NBLK_REG_MAX + NBLK_SPK_MAX

NW = 32
TPW = N_TOK // NW
ROWS_G = A_CAP // NW


def _router_body(x_ref, wr_ref, br_ref, lg_ref):
    res = jnp.dot(x_ref[...], wr_ref[...], preferred_element_type=jnp.float32)
    lg_ref[...] = res + br_ref[0:1, :]


def _phase1_body(lg_hbm, sp_hbm, ek_hbm, wts_hbm, cnts_hbm,
                 lgv, spv, ekv, wkv, cnt):
    c = lax.axis_index("c")
    s = lax.axis_index("s")
    w = s * 2 + c
    base_t = w * TPW
    lane = lax.broadcasted_iota(jnp.int32, (16,), 0)
    zero16i = jnp.zeros((16,), jnp.int32)
    zero16f = jnp.zeros((16,), jnp.float32)

    pltpu.sync_copy(lg_hbm.at[pl.ds(base_t, TPW)], lgv)
    pltpu.sync_copy(sp_hbm.at[pl.ds(base_t, TPW)], spv)

    def grp1_body(g, cntv):
        e0vec = zero16i
        e1vec = zero16i
        p0vec = zero16f
        p1vec = zero16f
        svec = zero16f
        for i in range(16):
            t = g * 16 + i
            lg = lgv[t, :]
            sp = spv[t, :]
            avg = jnp.sum(sp, axis=0) * (1.0 / SPIKE_LEN)
            valid = lane < TOTAL
            spike_m = jnp.logical_and(lane >= NUM_EXPERTS, valid)
            adj = lg + jnp.where(spike_m, avg, 0.0)
            adj = jnp.where(valid, adj, -1e30)
            mx = jnp.max(adj, axis=0)
            p = jnp.where(valid, jnp.exp(adj - mx), 0.0)
            skeys, svals = plsc.sort_key_val(p, lane, descending=True)
            e0 = svals[0]
            e1 = svals[1]
            sel = lane == i
            e0vec = jnp.where(sel, e0, e0vec)
            e1vec = jnp.where(sel, e1, e1vec)
            p0vec = jnp.where(sel, skeys[0], p0vec)
            p1vec = jnp.where(sel, skeys[1], p1vec)
            svec = jnp.where(sel, jnp.sum(p, axis=0), svec)
            cntv = cntv + (lane == e0).astype(jnp.int32)
            cntv = cntv + (lane == e1).astype(jnp.int32)
        pr0 = p0vec / svec
        pr1 = p1vec / svec
        den = pr0 + pr1 + 1e-9
        ekv[0, pl.ds(g * 16, 16)] = e0vec
        ekv[1, pl.ds(g * 16, 16)] = e1vec
        wkv[0, pl.ds(g * 16, 16)] = pr0 / den
        wkv[1, pl.ds(g * 16, 16)] = pr1 / den
        return cntv

    cntv = lax.fori_loop(0, TPW // 16, grp1_body, zero16i)
    cnt[...] = cntv
    pltpu.sync_copy(cnt, cnts_hbm.at[w])
    for k in range(2):
        pltpu.sync_copy(ekv.at[k], ek_hbm.at[pl.ds(k * N_TOK + base_t, TPW)])
        pltpu.sync_copy(wkv.at[k], wts_hbm.at[pl.ds(k * N_TOK + base_t, TPW)])


def _phase2_body(ek_hbm, cnts_hbm, meta_hbm, slots_hbm, perm_hbm,
                 allcnt, ekv, metav, slotsv, tokv, sem):
    c = lax.axis_index("c")
    s = lax.axis_index("s")
    w = s * 2 + c
    base_t = w * TPW
    lane = lax.broadcasted_iota(jnp.int32, (16,), 0)
    zero16i = jnp.zeros((16,), jnp.int32)

    pltpu.sync_copy(cnts_hbm, allcnt)
    tot = zero16i
    mybase = zero16i
    for r in range(NW):
        row = allcnt[r, :]
        tot = tot + row
        mybase = mybase + jnp.where(r < w, row, 0)
    tot = jnp.where(lane < TOTAL, tot, 0)
    padded = ((tot + (BT - 1)) // BT) * BT
    csum = plsc.cumsum(padded)
    off = csum - padded
    csum_blk = csum // BT
    nr = csum_blk[NUM_EXPERTS - 1]
    na = csum_blk[TOTAL - 1]
    ns = na - nr

    @pl.when(w == 0)
    def _meta():
        for i in range(8):
            metav[pl.ds(i * 16, 16)] = zero16i
        metav[pl.ds(112, 16)] = jnp.where(
            lane == 8, nr, jnp.where(lane == 9, ns, 0))
        for cch in range(3):
            b = cch * 16 + lane
            acc = zero16i
            for e in range(TOTAL):
                acc = acc + (b >= csum_blk[e]).astype(jnp.int32)
            is_reg = acc < NUM_EXPERTS
            val = jnp.where(is_reg, acc, acc - NUM_EXPERTS)
            pos = jnp.where(is_reg, b, 64 + b - nr)
            valid_b = b < na
            pos = jnp.where(valid_b, pos, 127)
            val = jnp.where(valid_b, val, 0)
            plsc.store_scatter(metav, [pos], val)
        pltpu.sync_copy(metav, meta_hbm)

    for k in range(2):
        pltpu.sync_copy(ek_hbm.at[pl.ds(k * N_TOK + base_t, TPW)], ekv.at[k])

    def grp3_body(g, mbv):
        e0vec = ekv[0, pl.ds(g * 16, 16)]
        e1vec = ekv[1, pl.ds(g * 16, 16)]
        slot0 = zero16i
        slot1 = zero16i
        for i in range(16):
            sel = lane == i
            e0 = e0vec[i]
            oh0 = lane == e0
            s0 = jnp.sum(jnp.where(oh0, mbv, 0), axis=0)
            mbv = mbv + oh0.astype(jnp.int32)
            e1 = e1vec[i]
            oh1 = lane == e1
            s1 = jnp.sum(jnp.where(oh1, mbv, 0), axis=0)
            mbv = mbv + oh1.astype(jnp.int32)
            slot0 = jnp.where(sel, s0, slot0)
            slot1 = jnp.where(sel, s1, slot1)
        slotsv[0, pl.ds(g * 16, 16)] = slot0
        slotsv[1, pl.ds(g * 16, 16)] = slot1
        return mbv

    lax.fori_loop(0, TPW // 16, grp3_body, off + mybase)
    for k in range(2):
        pltpu.sync_copy(slotsv.at[k],
                        slots_hbm.at[pl.ds(k * N_TOK + base_t, TPW)])
    for g in range(TPW // 16):
        tokv[pl.ds(g * 16, 16)] = base_t + g * 16 + lane
    for k in range(2):
        pltpu.async_copy(tokv, perm_hbm.at[slotsv.at[k]], sem).wait()


def _xgather_body(x_hbm, perm_hbm, xs_hbm, permv, buf0, buf1, sem0, sem1):
    c = lax.axis_index("c")
    s = lax.axis_index("s")
    w = s * 2 + c
    base = w * ROWS_G
    pltpu.sync_copy(perm_hbm.at[pl.ds(base, ROWS_G)], permv)
    for i in range(ROWS_G // 8 // 2):
        sl_ = pl.ds(i * 16, 16)
        v = permv[sl_]
        permv[sl_] = jnp.minimum(jnp.maximum(v, 0), N_TOK - 1)
    tail = pl.ds(ROWS_G - 16, 16)
    tv = permv[tail]
    permv[tail] = jnp.minimum(jnp.maximum(tv, 0), N_TOK - 1)
    chunks = [(i * 40, 40) for i in range(8)] + [(320, 24)]
    bufs = (buf0, buf1)
    sems = (sem0, sem1)
    handles = [None, None]

    def fire(i):
        off, ln = chunks[i]
        handles[i % 2] = pltpu.async_copy(
            x_hbm.at[permv.at[pl.ds(off, ln)]],
            bufs[i % 2].at[pl.ds(0, ln)], sems[i % 2])

    fire(0)
    for i in range(len(chunks)):
        if i + 1 < len(chunks):
            fire(i + 1)
        handles[i % 2].wait()
        off, ln = chunks[i]
        pltpu.sync_copy(bufs[i % 2].at[pl.ds(0, ln)],
                        xs_hbm.at[pl.ds(base + off, ln)])


def _experts_body(m_ref, xs_ref, w1r_ref, w2r_ref, b1r_ref, b2r_ref,
                  w1s_ref, w2s_ref, b1s_ref, b2s_ref, y_ref):
    j = pl.program_id(0)
    nr = m_ref[120]
    ns = m_ref[121]

    @pl.when(jnp.logical_and(j < NBLK_REG_MAX, j < nr))
    def _reg():
        x = xs_ref[...].astype(jnp.bfloat16)
        h = jnp.dot(x, w1r_ref[0], preferred_element_type=jnp.float32)
        h = jnp.maximum(h + b1r_ref[0], 0.0).astype(jnp.bfloat16)
        y_ref[...] = jnp.dot(h, w2r_ref[0],
                             preferred_element_type=jnp.float32) + b2r_ref[0]

    @pl.when(jnp.logical_and(j >= NBLK_REG_MAX, j - NBLK_REG_MAX < ns))
    def _spk():
        x = xs_ref[...].astype(jnp.bfloat16)
        h = jnp.dot(x, w1s_ref[0], preferred_element_type=jnp.float32)
        h = jnp.maximum(h + b1s_ref[0], 0.0).astype(jnp.bfloat16)
        y_ref[...] = jnp.dot(h, w2s_ref[0],
                             preferred_element_type=jnp.float32) + b2s_ref[0]


def _row_blk(j, m):
    return jnp.where(
        j < NBLK_REG_MAX,
        jnp.where(j < m[120], j, NBLK - 1),
        jnp.where(j - NBLK_REG_MAX < m[121],
                  m[120] + (j - NBLK_REG_MAX), NBLK - 1))


def _reg_e(j, m):
    return jnp.where(j < jnp.minimum(m[120], NBLK_REG_MAX), m[j], 0)


def _spk_e(j, m):
    return jnp.where(
        jnp.logical_and(j >= NBLK_REG_MAX, j - NBLK_REG_MAX < m[121]),
        m[64 + jnp.maximum(j - NBLK_REG_MAX, 0)], 0)


def _combine_body(y_hbm, slots_hbm, wts_hbm, out_hbm,
                  sl0, sl1, w0v, w1v, yb0, yb1, ob, sem0, sem1):
    c = lax.axis_index("c")
    s = lax.axis_index("s")
    w = s * 2 + c
    base = w * TPW
    pltpu.sync_copy(slots_hbm.at[pl.ds(base, TPW)], sl0)
    pltpu.sync_copy(slots_hbm.at[pl.ds(N_TOK + base, TPW)], sl1)
    pltpu.sync_copy(wts_hbm.at[pl.ds(base, TPW)], w0v)
    pltpu.sync_copy(wts_hbm.at[pl.ds(N_TOK + base, TPW)], w1v)
    def ch_body(ci, carry):
        h0 = pltpu.async_copy(y_hbm.at[sl0.at[pl.ds(ci * 16, 16)]], yb0, sem0)
        h1 = pltpu.async_copy(y_hbm.at[sl1.at[pl.ds(ci * 16, 16)]], yb1, sem1)
        h0.wait()
        h1.wait()
        w0vec = w0v[pl.ds(ci * 16, 16)]
        w1vec = w1v[pl.ds(ci * 16, 16)]
        for r in range(16):
            w0 = w0vec[r]
            w1 = w1vec[r]
            for sb in range(OUT_DIM // 16):
                sl_ = pl.ds(sb * 16, 16)
                ob[r, sl_] = yb0[r, sl_] * w0 + yb1[r, sl_] * w1
        pltpu.sync_copy(ob, out_hbm.at[pl.ds(base + ci * 16, 16)])
        return carry

    lax.fori_loop(0, TPW // 16, ch_body, 0)


def _pipeline(sequence_repr_with_baseline, spike_indicators, Wr, br,
              W1r, b1r, W2r, b2r, W1s, b1s, W2s, b2s):
    x = sequence_repr_with_baseline
    wr_pad = jnp.zeros((D_IN, 16), jnp.float32).at[:, :TOTAL].set(Wr)
    br_pad = jnp.zeros((8, 16), jnp.float32).at[:, :TOTAL].set(
        jnp.broadcast_to(br, (8, TOTAL)))

    logits = pl.pallas_call(
        _router_body,
        grid=(N_TOK // BT,),
        in_specs=[
            pl.BlockSpec((BT, D_IN), lambda i: (i, 0)),
            pl.BlockSpec((D_IN, 16), lambda i: (0, 0)),
            pl.BlockSpec((8, 16), lambda i: (0, 0)),
        ],
        out_specs=pl.BlockSpec((BT, 16), lambda i: (i, 0)),
        out_shape=jax.ShapeDtypeStruct((N_TOK, 16), jnp.float32),
    )(x, wr_pad, br_pad)

    mesh2 = plsc.VectorSubcoreMesh(core_axis_name="c", subcore_axis_name="s")
    ek, wts, cnts = pl.kernel(
        _phase1_body,
        out_type=[
            jax.ShapeDtypeStruct((2 * N_TOK,), jnp.int32),
            jax.ShapeDtypeStruct((2 * N_TOK,), jnp.float32),
            jax.ShapeDtypeStruct((NW, 16), jnp.int32),
        ],
        mesh=mesh2,
        compiler_params=pltpu.CompilerParams(needs_layout_passes=False),
        scratch_types=[
            pltpu.VMEM((TPW, 16), jnp.float32),
            pltpu.VMEM((TPW, 16), jnp.float32),
            pltpu.VMEM((2, TPW), jnp.int32),
            pltpu.VMEM((2, TPW), jnp.float32),
            pltpu.VMEM((16,), jnp.int32),
        ],
    )(logits, spike_indicators)

    meta, slots, perm = pl.kernel(
        _phase2_body,
        out_type=[
            jax.ShapeDtypeStruct((128,), jnp.int32),
            jax.ShapeDtypeStruct((2 * N_TOK,), jnp.int32),
            jax.ShapeDtypeStruct((A_CAP,), jnp.int32),
        ],
        mesh=mesh2,
        compiler_params=pltpu.CompilerParams(needs_layout_passes=False),
        scratch_types=[
            pltpu.VMEM((NW, 16), jnp.int32),
            pltpu.VMEM((2, TPW), jnp.int32),
            pltpu.VMEM((128,), jnp.int32),
            pltpu.VMEM((2, TPW), jnp.int32),
            pltpu.VMEM((TPW,), jnp.int32),
            pltpu.SemaphoreType.DMA,
        ],
    )(ek, cnts)
    xs = pl.kernel(
        _xgather_body,
        out_type=jax.ShapeDtypeStruct((A_CAP, D_IN), jnp.float32),
        mesh=mesh2,
        compiler_params=pltpu.CompilerParams(needs_layout_passes=False),
        scratch_types=[
            pltpu.VMEM((ROWS_G,), jnp.int32),
            pltpu.VMEM((40, D_IN), jnp.float32),
            pltpu.VMEM((40, D_IN), jnp.float32),
            pltpu.SemaphoreType.DMA,
            pltpu.SemaphoreType.DMA,
        ],
    )(x, perm)

    grid_spec = pltpu.PrefetchScalarGridSpec(
        num_scalar_prefetch=1,
        grid=(GRID_E,),
        in_specs=[
            pl.BlockSpec((BT, D_IN), lambda j, m: (_row_blk(j, m), 0)),
            pl.BlockSpec((1, D_IN, EXPERT_DIM), lambda j, m: (_reg_e(j, m), 0, 0)),
            pl.BlockSpec((1, EXPERT_DIM, OUT_DIM), lambda j, m: (_reg_e(j, m), 0, 0)),
            pl.BlockSpec((1, 1, EXPERT_DIM), lambda j, m: (_reg_e(j, m), 0, 0)),
            pl.BlockSpec((1, 1, OUT_DIM), lambda j, m: (_reg_e(j, m), 0, 0)),
            pl.BlockSpec((1, D_IN, SPIKE_CAP), lambda j, m: (_spk_e(j, m), 0, 0)),
            pl.BlockSpec((1, SPIKE_CAP, OUT_DIM), lambda j, m: (_spk_e(j, m), 0, 0)),
            pl.BlockSpec((1, 1, SPIKE_CAP), lambda j, m: (_spk_e(j, m), 0, 0)),
            pl.BlockSpec((1, 1, OUT_DIM), lambda j, m: (_spk_e(j, m), 0, 0)),
        ],
        out_specs=pl.BlockSpec((BT, OUT_DIM), lambda j, m: (_row_blk(j, m), 0)),
    )
    y = pl.pallas_call(
        _experts_body,
        grid_spec=grid_spec,
        out_shape=jax.ShapeDtypeStruct((A_CAP, OUT_DIM), jnp.float32),
    )(meta, xs,
      W1r.astype(jnp.bfloat16), W2r.astype(jnp.bfloat16),
      b1r[:, None, :], b2r[:, None, :],
      W1s.astype(jnp.bfloat16), W2s.astype(jnp.bfloat16),
      b1s[:, None, :], b2s[:, None, :])

    out = pl.kernel(
        _combine_body,
        out_type=jax.ShapeDtypeStruct((N_TOK, OUT_DIM), jnp.float32),
        mesh=mesh2,
        compiler_params=pltpu.CompilerParams(needs_layout_passes=False),
        scratch_types=[
            pltpu.VMEM((TPW,), jnp.int32),
            pltpu.VMEM((TPW,), jnp.int32),
            pltpu.VMEM((TPW,), jnp.float32),
            pltpu.VMEM((TPW,), jnp.float32),
            pltpu.VMEM((16, OUT_DIM), jnp.float32),
            pltpu.VMEM((16, OUT_DIM), jnp.float32),
            pltpu.VMEM((16, OUT_DIM), jnp.float32),
            pltpu.SemaphoreType.DMA,
            pltpu.SemaphoreType.DMA,
        ],
    )(y, slots, wts)
    return logits, meta, slots, wts, perm, xs, y, out


def kernel(sequence_repr_with_baseline, spike_indicators, Wr, br,
           W1r, b1r, W2r, b2r, W1s, b1s, W2s, b2s):
    return _pipeline(sequence_repr_with_baseline, spike_indicators, Wr, br,
                     W1r, b1r, W2r, b2r, W1s, b1s, W2s, b2s)[-1]


# speedup vs baseline: 1.9300x; 1.9300x over previous
import functools

import jax
import jax.numpy as jnp
from jax import lax
from jax.experimental import pallas as pl
from jax.experimental.pallas import tpu as pltpu
from jax.experimental.pallas import tpu_sc as plsc

N_TOK = 4096
D_IN = 768
NUM_EXPERTS = 8
NUM_SPIKE = 2
TOTAL = NUM_EXPERTS + NUM_SPIKE
EXPERT_DIM = 768
SPIKE_CAP = 1536
OUT_DIM = 768
SPIKE_LEN = 16

BT = 256
NBLK_REG_MAX = 40
NBLK_SPK_MAX = 34
NBLK = 43
A_CAP = NBLK * BT
GRID_E = NBLK_REG_MAX + NBLK_SPK_MAX

NW = 32
TPW = N_TOK // NW
ROWS_G = A_CAP // NW


def _router_body(x_ref, wr_ref, br_ref, lg_ref):
    res = jnp.dot(x_ref[...], wr_ref[...], preferred_element_type=jnp.float32)
    lg_ref[...] = res + br_ref[0:1, :]


def _phase1_body(lg_hbm, sp_hbm, ek_hbm, wts_hbm, cnts_hbm,
                 lgv, spv, ekv, wkv, cnt):
    c = lax.axis_index("c")
    s = lax.axis_index("s")
    w = s * 2 + c
    base_t = w * TPW
    lane = lax.broadcasted_iota(jnp.int32, (16,), 0)
    zero16i = jnp.zeros((16,), jnp.int32)
    zero16f = jnp.zeros((16,), jnp.float32)

    pltpu.sync_copy(lg_hbm.at[pl.ds(base_t, TPW)], lgv)
    pltpu.sync_copy(sp_hbm.at[pl.ds(base_t, TPW)], spv)

    def grp1_body(g, cntv):
        e0vec = zero16i
        e1vec = zero16i
        p0vec = zero16f
        p1vec = zero16f
        svec = zero16f
        for i in range(16):
            t = g * 16 + i
            lg = lgv[t, :]
            sp = spv[t, :]
            avg = jnp.sum(sp, axis=0) * (1.0 / SPIKE_LEN)
            valid = lane < TOTAL
            spike_m = jnp.logical_and(lane >= NUM_EXPERTS, valid)
            adj = lg + jnp.where(spike_m, avg, 0.0)
            adj = jnp.where(valid, adj, -1e30)
            mx = jnp.max(adj, axis=0)
            p = jnp.where(valid, jnp.exp(adj - mx), 0.0)
            skeys, svals = plsc.sort_key_val(p, lane, descending=True)
            e0 = svals[0]
            e1 = svals[1]
            sel = lane == i
            e0vec = jnp.where(sel, e0, e0vec)
            e1vec = jnp.where(sel, e1, e1vec)
            p0vec = jnp.where(sel, skeys[0], p0vec)
            p1vec = jnp.where(sel, skeys[1], p1vec)
            svec = jnp.where(sel, jnp.sum(p, axis=0), svec)
            cntv = cntv + (lane == e0).astype(jnp.int32)
            cntv = cntv + (lane == e1).astype(jnp.int32)
        pr0 = p0vec / svec
        pr1 = p1vec / svec
        den = pr0 + pr1 + 1e-9
        ekv[0, pl.ds(g * 16, 16)] = e0vec
        ekv[1, pl.ds(g * 16, 16)] = e1vec
        wkv[0, pl.ds(g * 16, 16)] = pr0 / den
        wkv[1, pl.ds(g * 16, 16)] = pr1 / den
        return cntv

    cntv = lax.fori_loop(0, TPW // 16, grp1_body, zero16i)
    cnt[...] = cntv
    pltpu.sync_copy(cnt, cnts_hbm.at[w])
    for k in range(2):
        pltpu.sync_copy(ekv.at[k], ek_hbm.at[pl.ds(k * N_TOK + base_t, TPW)])
        pltpu.sync_copy(wkv.at[k], wts_hbm.at[pl.ds(k * N_TOK + base_t, TPW)])


def _phase2_body(ek_hbm, cnts_hbm, x_hbm, meta_hbm, slots_hbm, xs_hbm,
                 allcnt, ekv, metav, slotsv, xbufs, sem):
    c = lax.axis_index("c")
    s = lax.axis_index("s")
    w = s * 2 + c
    base_t = w * TPW
    lane = lax.broadcasted_iota(jnp.int32, (16,), 0)
    zero16i = jnp.zeros((16,), jnp.int32)

    pltpu.sync_copy(cnts_hbm, allcnt)
    tot = zero16i
    mybase = zero16i
    for r in range(NW):
        row = allcnt[r, :]
        tot = tot + row
        mybase = mybase + jnp.where(r < w, row, 0)
    tot = jnp.where(lane < TOTAL, tot, 0)
    padded = ((tot + (BT - 1)) // BT) * BT
    csum = plsc.cumsum(padded)
    off = csum - padded
    csum_blk = csum // BT
    nr = csum_blk[NUM_EXPERTS - 1]
    na = csum_blk[TOTAL - 1]
    ns = na - nr

    @pl.when(w == 0)
    def _meta():
        for i in range(8):
            metav[pl.ds(i * 16, 16)] = zero16i
        metav[pl.ds(112, 16)] = jnp.where(
            lane == 8, nr, jnp.where(lane == 9, ns, 0))
        for cch in range(3):
            b = cch * 16 + lane
            acc = zero16i
            for e in range(TOTAL):
                acc = acc + (b >= csum_blk[e]).astype(jnp.int32)
            is_reg = acc < NUM_EXPERTS
            val = jnp.where(is_reg, acc, acc - NUM_EXPERTS)
            pos = jnp.where(is_reg, b, 64 + b - nr)
            valid_b = b < na
            pos = jnp.where(valid_b, pos, 127)
            val = jnp.where(valid_b, val, 0)
            plsc.store_scatter(metav, [pos], val)
        pltpu.sync_copy(metav, meta_hbm)

    for k in range(2):
        pltpu.sync_copy(ek_hbm.at[pl.ds(k * N_TOK + base_t, TPW)], ekv.at[k])

    def grp3_body(g, mbv):
        e0vec = ekv[0, pl.ds(g * 16, 16)]
        e1vec = ekv[1, pl.ds(g * 16, 16)]
        slot0 = zero16i
        slot1 = zero16i
        for i in range(16):
            sel = lane == i
            e0 = e0vec[i]
            oh0 = lane == e0
            s0 = jnp.sum(jnp.where(oh0, mbv, 0), axis=0)
            mbv = mbv + oh0.astype(jnp.int32)
            e1 = e1vec[i]
            oh1 = lane == e1
            s1 = jnp.sum(jnp.where(oh1, mbv, 0), axis=0)
            mbv = mbv + oh1.astype(jnp.int32)
            slot0 = jnp.where(sel, s0, slot0)
            slot1 = jnp.where(sel, s1, slot1)
        hi = g // 4
        lo = (g - hi * 4) * 16
        slotsv[hi, pl.ds(lo, 16)] = slot0
        slotsv[2 + hi, pl.ds(lo, 16)] = slot1
        return mbv

    lax.fori_loop(0, TPW // 16, grp3_body, off + mybase)
    for k in range(2):
        for hh in range(2):
            pltpu.sync_copy(
                slotsv.at[k * 2 + hh],
                slots_hbm.at[pl.ds(k * N_TOK + base_t + hh * 64, 64)])
    pltpu.sync_copy(x_hbm.at[pl.ds(base_t, TPW)], xbufs)
    handles = []
    for k in range(2):
        for hh in range(2):
            handles.append(pltpu.async_copy(
                xbufs.at[pl.ds(hh * 64, 64)],
                xs_hbm.at[slotsv.at[k * 2 + hh]], sem))
    for h in handles:
        h.wait()


def _experts_body(m_ref, xs_ref, w1r_ref, w2r_ref, b1r_ref, b2r_ref,
                  w1s_ref, w2s_ref, b1s_ref, b2s_ref, y_ref):
    j = pl.program_id(0)
    nr = m_ref[120]
    ns = m_ref[121]

    @pl.when(jnp.logical_and(j < NBLK_REG_MAX, j < nr))
    def _reg():
        x = xs_ref[...].astype(jnp.bfloat16)
        h = jnp.dot(x, w1r_ref[0], preferred_element_type=jnp.float32)
        h = jnp.maximum(h + b1r_ref[0], 0.0).astype(jnp.bfloat16)
        y_ref[...] = jnp.dot(h, w2r_ref[0],
                             preferred_element_type=jnp.float32) + b2r_ref[0]

    @pl.when(jnp.logical_and(j >= NBLK_REG_MAX, j - NBLK_REG_MAX < ns))
    def _spk():
        x = xs_ref[...].astype(jnp.bfloat16)
        h = jnp.dot(x, w1s_ref[0], preferred_element_type=jnp.float32)
        h = jnp.maximum(h + b1s_ref[0], 0.0).astype(jnp.bfloat16)
        y_ref[...] = jnp.dot(h, w2s_ref[0],
                             preferred_element_type=jnp.float32) + b2s_ref[0]


def _row_blk(j, m):
    return jnp.where(
        j < NBLK_REG_MAX,
        jnp.where(j < m[120], j, NBLK - 1),
        jnp.where(j - NBLK_REG_MAX < m[121],
                  m[120] + (j - NBLK_REG_MAX), NBLK - 1))


def _reg_e(j, m):
    return jnp.where(j < jnp.minimum(m[120], NBLK_REG_MAX), m[j], 0)


def _spk_e(j, m):
    return jnp.where(
        jnp.logical_and(j >= NBLK_REG_MAX, j - NBLK_REG_MAX < m[121]),
        m[64 + jnp.maximum(j - NBLK_REG_MAX, 0)], 0)


def _combine_body(y_hbm, slots_hbm, wts_hbm, out_hbm,
                  sl0, sl1, w0v, w1v, yb0a, yb1a, yb0b, yb1b, ob,
                  sem0a, sem1a, sem0b, sem1b):
    c = lax.axis_index("c")
    s = lax.axis_index("s")
    w = s * 2 + c
    base = w * TPW
    pltpu.sync_copy(slots_hbm.at[pl.ds(base, TPW)], sl0)
    pltpu.sync_copy(slots_hbm.at[pl.ds(N_TOK + base, TPW)], sl1)
    pltpu.sync_copy(wts_hbm.at[pl.ds(base, TPW)], w0v)
    pltpu.sync_copy(wts_hbm.at[pl.ds(N_TOK + base, TPW)], w1v)
    NCH = TPW // 16
    pairs = ((yb0a, yb1a, sem0a, sem1a), (yb0b, yb1b, sem0b, sem1b))

    def fire(ci, pb):
        b0, b1, s0_, s1_ = pairs[pb]
        pltpu.async_copy(y_hbm.at[sl0.at[pl.ds(ci * 16, 16)]], b0, s0_)
        pltpu.async_copy(y_hbm.at[sl1.at[pl.ds(ci * 16, 16)]], b1, s1_)

    def drain(pb):
        b0, b1, s0_, s1_ = pairs[pb]
        pltpu.make_async_copy(y_hbm.at[sl0.at[pl.ds(0, 16)]], b0, s0_).wait()
        pltpu.make_async_copy(y_hbm.at[sl1.at[pl.ds(0, 16)]], b1, s1_).wait()

    def compute(ci, pb):
        b0, b1, _, _ = pairs[pb]
        w0vec = w0v[pl.ds(ci * 16, 16)]
        w1vec = w1v[pl.ds(ci * 16, 16)]
        for r in range(16):
            w0 = w0vec[r]
            w1 = w1vec[r]
            for sb in range(OUT_DIM // 16):
                sl_ = pl.ds(sb * 16, 16)
                ob[r, sl_] = b0[r, sl_] * w0 + b1[r, sl_] * w1
        pltpu.sync_copy(ob, out_hbm.at[pl.ds(base + ci * 16, 16)])

    fire(0, 0)

    def ring_body(g, carry):
        ci0 = g * 2

        @pl.when(ci0 + 1 < NCH)
        def _f1():
            fire(ci0 + 1, 1)

        drain(0)
        compute(ci0, 0)

        @pl.when(ci0 + 2 < NCH)
        def _f2():
            fire(ci0 + 2, 0)

        drain(1)
        compute(ci0 + 1, 1)
        return carry

    lax.fori_loop(0, NCH // 2, ring_body, 0)


def _pipeline(sequence_repr_with_baseline, spike_indicators, Wr, br,
              W1r, b1r, W2r, b2r, W1s, b1s, W2s, b2s):
    x = sequence_repr_with_baseline
    wr_pad = jnp.zeros((D_IN, 16), jnp.float32).at[:, :TOTAL].set(Wr)
    br_pad = jnp.zeros((8, 16), jnp.float32).at[:, :TOTAL].set(
        jnp.broadcast_to(br, (8, TOTAL)))

    logits = pl.pallas_call(
        _router_body,
        grid=(N_TOK // BT,),
        in_specs=[
            pl.BlockSpec((BT, D_IN), lambda i: (i, 0)),
            pl.BlockSpec((D_IN, 16), lambda i: (0, 0)),
            pl.BlockSpec((8, 16), lambda i: (0, 0)),
        ],
        out_specs=pl.BlockSpec((BT, 16), lambda i: (i, 0)),
        out_shape=jax.ShapeDtypeStruct((N_TOK, 16), jnp.float32),
    )(x, wr_pad, br_pad)

    mesh2 = plsc.VectorSubcoreMesh(core_axis_name="c", subcore_axis_name="s")
    ek, wts, cnts = pl.kernel(
        _phase1_body,
        out_type=[
            jax.ShapeDtypeStruct((2 * N_TOK,), jnp.int32),
            jax.ShapeDtypeStruct((2 * N_TOK,), jnp.float32),
            jax.ShapeDtypeStruct((NW, 16), jnp.int32),
        ],
        mesh=mesh2,
        compiler_params=pltpu.CompilerParams(needs_layout_passes=False),
        scratch_types=[
            pltpu.VMEM((TPW, 16), jnp.float32),
            pltpu.VMEM((TPW, 16), jnp.float32),
            pltpu.VMEM((2, TPW), jnp.int32),
            pltpu.VMEM((2, TPW), jnp.float32),
            pltpu.VMEM((16,), jnp.int32),
        ],
    )(logits, spike_indicators)

    meta, slots, xs = pl.kernel(
        _phase2_body,
        out_type=[
            jax.ShapeDtypeStruct((128,), jnp.int32),
            jax.ShapeDtypeStruct((2 * N_TOK,), jnp.int32),
            jax.ShapeDtypeStruct((A_CAP, D_IN), jnp.float32),
        ],
        mesh=mesh2,
        compiler_params=pltpu.CompilerParams(needs_layout_passes=False),
        scratch_types=[
            pltpu.VMEM((NW, 16), jnp.int32),
            pltpu.VMEM((2, TPW), jnp.int32),
            pltpu.VMEM((128,), jnp.int32),
            pltpu.VMEM((4, 64), jnp.int32),
            pltpu.VMEM((TPW, D_IN), jnp.float32),
            pltpu.SemaphoreType.DMA,
        ],
    )(ek, cnts, x)

    grid_spec = pltpu.PrefetchScalarGridSpec(
        num_scalar_prefetch=1,
        grid=(GRID_E,),
        in_specs=[
            pl.BlockSpec((BT, D_IN), lambda j, m: (_row_blk(j, m), 0)),
            pl.BlockSpec((1, D_IN, EXPERT_DIM), lambda j, m: (_reg_e(j, m), 0, 0)),
            pl.BlockSpec((1, EXPERT_DIM, OUT_DIM), lambda j, m: (_reg_e(j, m), 0, 0)),
            pl.BlockSpec((1, 1, EXPERT_DIM), lambda j, m: (_reg_e(j, m), 0, 0)),
            pl.BlockSpec((1, 1, OUT_DIM), lambda j, m: (_reg_e(j, m), 0, 0)),
            pl.BlockSpec((1, D_IN, SPIKE_CAP), lambda j, m: (_spk_e(j, m), 0, 0)),
            pl.BlockSpec((1, SPIKE_CAP, OUT_DIM), lambda j, m: (_spk_e(j, m), 0, 0)),
            pl.BlockSpec((1, 1, SPIKE_CAP), lambda j, m: (_spk_e(j, m), 0, 0)),
            pl.BlockSpec((1, 1, OUT_DIM), lambda j, m: (_spk_e(j, m), 0, 0)),
        ],
        out_specs=pl.BlockSpec((BT, OUT_DIM), lambda j, m: (_row_blk(j, m), 0)),
    )
    y = pl.pallas_call(
        _experts_body,
        grid_spec=grid_spec,
        out_shape=jax.ShapeDtypeStruct((A_CAP, OUT_DIM), jnp.float32),
    )(meta, xs,
      W1r.astype(jnp.bfloat16), W2r.astype(jnp.bfloat16),
      b1r[:, None, :], b2r[:, None, :],
      W1s.astype(jnp.bfloat16), W2s.astype(jnp.bfloat16),
      b1s[:, None, :], b2s[:, None, :])

    out = pl.kernel(
        _combine_body,
        out_type=jax.ShapeDtypeStruct((N_TOK, OUT_DIM), jnp.float32),
        mesh=mesh2,
        compiler_params=pltpu.CompilerParams(needs_layout_passes=False),
        scratch_types=[
            pltpu.VMEM((TPW,), jnp.int32),
            pltpu.VMEM((TPW,), jnp.int32),
            pltpu.VMEM((TPW,), jnp.float32),
            pltpu.VMEM((TPW,), jnp.float32),
            pltpu.VMEM((16, OUT_DIM), jnp.float32),
            pltpu.VMEM((16, OUT_DIM), jnp.float32),
            pltpu.VMEM((16, OUT_DIM), jnp.float32),
            pltpu.VMEM((16, OUT_DIM), jnp.float32),
            pltpu.VMEM((16, OUT_DIM), jnp.float32),
            pltpu.SemaphoreType.DMA,
            pltpu.SemaphoreType.DMA,
            pltpu.SemaphoreType.DMA,
            pltpu.SemaphoreType.DMA,
        ],
    )(y, slots, wts)
    return logits, meta, slots, wts, xs, y, out


def kernel(sequence_repr_with_baseline, spike_indicators, Wr, br,
           W1r, b1r, W2r, b2r, W1s, b1s, W2s, b2s):
    return _pipeline(sequence_repr_with_baseline, spike_indicators, Wr, br,
                     W1r, b1r, W2r, b2r, W1s, b1s, W2s, b2s)[-1]


# speedup vs baseline: 1.9665x; 1.0189x over previous
import functools

import jax
import jax.numpy as jnp
from jax import lax
from jax.experimental import pallas as pl
from jax.experimental.pallas import tpu as pltpu
from jax.experimental.pallas import tpu_sc as plsc

N_TOK = 4096
D_IN = 768
NUM_EXPERTS = 8
NUM_SPIKE = 2
TOTAL = NUM_EXPERTS + NUM_SPIKE
EXPERT_DIM = 768
SPIKE_CAP = 1536
OUT_DIM = 768
SPIKE_LEN = 16

BT = 256
NBLK_REG_MAX = 40
NBLK_SPK_MAX = 34
NBLK = 43
A_CAP = NBLK * BT
GRID_E = NBLK - 1

NW = 32
TPW = N_TOK // NW
ROWS_G = A_CAP // NW


def _router_body(x_ref, wr_ref, br_ref, lg_ref):
    res = jnp.dot(x_ref[...], wr_ref[...], preferred_element_type=jnp.float32)
    lg_ref[...] = res + br_ref[0:1, :]


def _phase1_body(lg_hbm, sp_hbm, ek_hbm, wts_hbm, cnts_hbm,
                 lgv, spv, ekv, wkv, cnt):
    c = lax.axis_index("c")
    s = lax.axis_index("s")
    w = s * 2 + c
    base_t = w * TPW
    lane = lax.broadcasted_iota(jnp.int32, (16,), 0)
    zero16i = jnp.zeros((16,), jnp.int32)
    zero16f = jnp.zeros((16,), jnp.float32)

    pltpu.sync_copy(lg_hbm.at[pl.ds(base_t, TPW)], lgv)
    pltpu.sync_copy(sp_hbm.at[pl.ds(base_t, TPW)], spv)

    def grp1_body(g, cntv):
        e0vec = zero16i
        e1vec = zero16i
        p0vec = zero16f
        p1vec = zero16f
        svec = zero16f
        for i in range(16):
            t = g * 16 + i
            lg = lgv[t, :]
            sp = spv[t, :]
            avg = jnp.sum(sp, axis=0) * (1.0 / SPIKE_LEN)
            valid = lane < TOTAL
            spike_m = jnp.logical_and(lane >= NUM_EXPERTS, valid)
            adj = lg + jnp.where(spike_m, avg, 0.0)
            adj = jnp.where(valid, adj, -1e30)
            mx = jnp.max(adj, axis=0)
            p = jnp.where(valid, jnp.exp(adj - mx), 0.0)
            skeys, svals = plsc.sort_key_val(p, lane, descending=True)
            e0 = svals[0]
            e1 = svals[1]
            sel = lane == i
            e0vec = jnp.where(sel, e0, e0vec)
            e1vec = jnp.where(sel, e1, e1vec)
            p0vec = jnp.where(sel, skeys[0], p0vec)
            p1vec = jnp.where(sel, skeys[1], p1vec)
            svec = jnp.where(sel, jnp.sum(p, axis=0), svec)
            cntv = cntv + (lane == e0).astype(jnp.int32)
            cntv = cntv + (lane == e1).astype(jnp.int32)
        pr0 = p0vec / svec
        pr1 = p1vec / svec
        den = pr0 + pr1 + 1e-9
        ekv[0, pl.ds(g * 16, 16)] = e0vec
        ekv[1, pl.ds(g * 16, 16)] = e1vec
        wkv[0, pl.ds(g * 16, 16)] = pr0 / den
        wkv[1, pl.ds(g * 16, 16)] = pr1 / den
        return cntv

    cntv = lax.fori_loop(0, TPW // 16, grp1_body, zero16i)
    cnt[...] = cntv
    pltpu.sync_copy(cnt, cnts_hbm.at[w])
    for k in range(2):
        pltpu.sync_copy(ekv.at[k], ek_hbm.at[pl.ds(k * N_TOK + base_t, TPW)])
        pltpu.sync_copy(wkv.at[k], wts_hbm.at[pl.ds(k * N_TOK + base_t, TPW)])


def _phase2_body(ek_hbm, cnts_hbm, x_hbm, meta_hbm, slots_hbm, xs_hbm,
                 allcnt, ekv, metav, slotsv, xbufs, sem):
    c = lax.axis_index("c")
    s = lax.axis_index("s")
    w = s * 2 + c
    base_t = w * TPW
    lane = lax.broadcasted_iota(jnp.int32, (16,), 0)
    zero16i = jnp.zeros((16,), jnp.int32)

    pltpu.sync_copy(cnts_hbm, allcnt)
    tot = zero16i
    mybase = zero16i
    for r in range(NW):
        row = allcnt[r, :]
        tot = tot + row
        mybase = mybase + jnp.where(r < w, row, 0)
    tot = jnp.where(lane < TOTAL, tot, 0)
    padded = ((tot + (BT - 1)) // BT) * BT
    csum = plsc.cumsum(padded)
    off = csum - padded
    csum_blk = csum // BT
    na = csum_blk[TOTAL - 1]

    @pl.when(w == 0)
    def _meta():
        for i in range(8):
            metav[pl.ds(i * 16, 16)] = zero16i
        metav[pl.ds(112, 16)] = jnp.where(lane == 8, na, 0)
        for cch in range(3):
            b = cch * 16 + lane
            acc = zero16i
            for e in range(TOTAL):
                acc = acc + (b >= csum_blk[e]).astype(jnp.int32)
            valid_b = b < na
            pos = jnp.where(valid_b, b, 127)
            val = jnp.where(valid_b, acc, 0)
            plsc.store_scatter(metav, [pos], val)
        pltpu.sync_copy(metav, meta_hbm)

    for k in range(2):
        pltpu.sync_copy(ek_hbm.at[pl.ds(k * N_TOK + base_t, TPW)], ekv.at[k])

    def grp3_body(g, mbv):
        e0vec = ekv[0, pl.ds(g * 16, 16)]
        e1vec = ekv[1, pl.ds(g * 16, 16)]
        slot0 = zero16i
        slot1 = zero16i
        for i in range(16):
            sel = lane == i
            e0 = e0vec[i]
            oh0 = lane == e0
            s0 = jnp.sum(jnp.where(oh0, mbv, 0), axis=0)
            mbv = mbv + oh0.astype(jnp.int32)
            e1 = e1vec[i]
            oh1 = lane == e1
            s1 = jnp.sum(jnp.where(oh1, mbv, 0), axis=0)
            mbv = mbv + oh1.astype(jnp.int32)
            slot0 = jnp.where(sel, s0, slot0)
            slot1 = jnp.where(sel, s1, slot1)
        hi = g // 2
        lo = (g - hi * 2) * 16
        slotsv[hi, pl.ds(lo, 16)] = slot0
        slotsv[4 + hi, pl.ds(lo, 16)] = slot1
        return mbv

    lax.fori_loop(0, TPW // 16, grp3_body, off + mybase)
    for k in range(2):
        for hh in range(4):
            pltpu.sync_copy(
                slotsv.at[k * 4 + hh],
                slots_hbm.at[pl.ds(k * N_TOK + base_t + hh * 32, 32)])
    pltpu.sync_copy(x_hbm.at[pl.ds(base_t, TPW)], xbufs)
    handles = []
    for k in range(2):
        for hh in range(4):
            handles.append(pltpu.async_copy(
                xbufs.at[pl.ds(hh * 32, 32)],
                xs_hbm.at[slotsv.at[k * 4 + hh]], sem))
    for h in handles:
        h.wait()


def _experts_body(m_ref, xs_ref, w1r_ref, w2r_ref, b1r_ref, b2r_ref,
                  w1s_ref, w2s_ref, b1s_ref, b2s_ref, y_ref):
    j = pl.program_id(0)
    na = m_ref[120]
    e_blk = m_ref[j]

    @pl.when(jnp.logical_and(j < na, e_blk < NUM_EXPERTS))
    def _reg():
        x = xs_ref[...].astype(jnp.bfloat16)
        h = jnp.dot(x, w1r_ref[0], preferred_element_type=jnp.float32)
        h = jnp.maximum(h + b1r_ref[0], 0.0).astype(jnp.bfloat16)
        y_ref[...] = jnp.dot(h, w2r_ref[0],
                             preferred_element_type=jnp.float32) + b2r_ref[0]

    @pl.when(jnp.logical_and(j < na, e_blk >= NUM_EXPERTS))
    def _spk():
        x = xs_ref[...].astype(jnp.bfloat16)
        h = jnp.dot(x, w1s_ref[0], preferred_element_type=jnp.float32)
        h = jnp.maximum(h + b1s_ref[0], 0.0).astype(jnp.bfloat16)
        y_ref[...] = jnp.dot(h, w2s_ref[0],
                             preferred_element_type=jnp.float32) + b2s_ref[0]


def _row_blk(j, m):
    return jnp.where(j < m[120], j, NBLK - 1)


def _reg_e(j, m):
    e = jnp.where(j < m[120], m[j], 0)
    return jnp.where(e < NUM_EXPERTS, e, 0)


def _spk_e(j, m):
    e = jnp.where(j < m[120], m[j], 0)
    return jnp.maximum(e - NUM_EXPERTS, 0)


def _combine_body(y_hbm, slots_hbm, wts_hbm, out_hbm,
                  sl0, sl1, w0v, w1v, *bufs_and_sems):
    ybufs = bufs_and_sems[:8]
    ob = bufs_and_sems[8]
    sems = bufs_and_sems[9:17]
    c = lax.axis_index("c")
    s = lax.axis_index("s")
    w = s * 2 + c
    base = w * TPW
    pltpu.sync_copy(slots_hbm.at[pl.ds(base, TPW)], sl0)
    pltpu.sync_copy(slots_hbm.at[pl.ds(N_TOK + base, TPW)], sl1)
    pltpu.sync_copy(wts_hbm.at[pl.ds(base, TPW)], w0v)
    pltpu.sync_copy(wts_hbm.at[pl.ds(N_TOK + base, TPW)], w1v)
    NCH = TPW // 8
    pairs = tuple((ybufs[2 * p], ybufs[2 * p + 1], sems[2 * p], sems[2 * p + 1])
                  for p in range(4))

    def fire(ci, pb):
        b0, b1, s0_, s1_ = pairs[pb]
        pltpu.async_copy(y_hbm.at[sl0.at[pl.ds(ci * 8, 8)]], b0, s0_)
        pltpu.async_copy(y_hbm.at[sl1.at[pl.ds(ci * 8, 8)]], b1, s1_)

    def drain(pb):
        b0, b1, s0_, s1_ = pairs[pb]
        pltpu.make_async_copy(y_hbm.at[sl0.at[pl.ds(0, 8)]], b0, s0_).wait()
        pltpu.make_async_copy(y_hbm.at[sl1.at[pl.ds(0, 8)]], b1, s1_).wait()

    def compute(ci, pb, half):
        b0, b1, _, _ = pairs[pb]
        w0vec = w0v[pl.ds((ci - half) * 8, 16)]
        w1vec = w1v[pl.ds((ci - half) * 8, 16)]
        for r in range(8):
            w0 = w0vec[half * 8 + r]
            w1 = w1vec[half * 8 + r]
            for sb in range(OUT_DIM // 16):
                sl_ = pl.ds(sb * 16, 16)
                ob[r, sl_] = b0[r, sl_] * w0 + b1[r, sl_] * w1
        pltpu.sync_copy(ob, out_hbm.at[pl.ds(base + ci * 8, 8)])

    for p in range(3):
        fire(p, p)

    def ring_body(g, carry):
        for b in range(4):
            ci = g * 4 + b

            @pl.when(ci + 3 < NCH)
            def _f(ci=ci, b=b):
                fire(ci + 3, (b + 3) % 4)

            drain(b)
            compute(ci, b, b % 2)
        return carry

    lax.fori_loop(0, NCH // 4, ring_body, 0)


def _pipeline(sequence_repr_with_baseline, spike_indicators, Wr, br,
              W1r, b1r, W2r, b2r, W1s, b1s, W2s, b2s):
    x = sequence_repr_with_baseline
    wr_pad = jnp.zeros((D_IN, 16), jnp.float32).at[:, :TOTAL].set(Wr)
    br_pad = jnp.zeros((8, 16), jnp.float32).at[:, :TOTAL].set(
        jnp.broadcast_to(br, (8, TOTAL)))

    logits = pl.pallas_call(
        _router_body,
        grid=(N_TOK // BT,),
        in_specs=[
            pl.BlockSpec((BT, D_IN), lambda i: (i, 0)),
            pl.BlockSpec((D_IN, 16), lambda i: (0, 0)),
            pl.BlockSpec((8, 16), lambda i: (0, 0)),
        ],
        out_specs=pl.BlockSpec((BT, 16), lambda i: (i, 0)),
        out_shape=jax.ShapeDtypeStruct((N_TOK, 16), jnp.float32),
    )(x, wr_pad, br_pad)

    mesh2 = plsc.VectorSubcoreMesh(core_axis_name="c", subcore_axis_name="s")
    ek, wts, cnts = pl.kernel(
        _phase1_body,
        out_type=[
            jax.ShapeDtypeStruct((2 * N_TOK,), jnp.int32),
            jax.ShapeDtypeStruct((2 * N_TOK,), jnp.float32),
            jax.ShapeDtypeStruct((NW, 16), jnp.int32),
        ],
        mesh=mesh2,
        compiler_params=pltpu.CompilerParams(needs_layout_passes=False),
        scratch_types=[
            pltpu.VMEM((TPW, 16), jnp.float32),
            pltpu.VMEM((TPW, 16), jnp.float32),
            pltpu.VMEM((2, TPW), jnp.int32),
            pltpu.VMEM((2, TPW), jnp.float32),
            pltpu.VMEM((16,), jnp.int32),
        ],
    )(logits, spike_indicators)

    meta, slots, xs = pl.kernel(
        _phase2_body,
        out_type=[
            jax.ShapeDtypeStruct((128,), jnp.int32),
            jax.ShapeDtypeStruct((2 * N_TOK,), jnp.int32),
            jax.ShapeDtypeStruct((A_CAP, D_IN), jnp.float32),
        ],
        mesh=mesh2,
        compiler_params=pltpu.CompilerParams(needs_layout_passes=False),
        scratch_types=[
            pltpu.VMEM((NW, 16), jnp.int32),
            pltpu.VMEM((2, TPW), jnp.int32),
            pltpu.VMEM((128,), jnp.int32),
            pltpu.VMEM((8, 32), jnp.int32),
            pltpu.VMEM((TPW, D_IN), jnp.float32),
            pltpu.SemaphoreType.DMA,
        ],
    )(ek, cnts, x)

    grid_spec = pltpu.PrefetchScalarGridSpec(
        num_scalar_prefetch=1,
        grid=(GRID_E,),
        in_specs=[
            pl.BlockSpec((BT, D_IN), lambda j, m: (_row_blk(j, m), 0)),
            pl.BlockSpec((1, D_IN, EXPERT_DIM), lambda j, m: (_reg_e(j, m), 0, 0)),
            pl.BlockSpec((1, EXPERT_DIM, OUT_DIM), lambda j, m: (_reg_e(j, m), 0, 0)),
            pl.BlockSpec((1, 1, EXPERT_DIM), lambda j, m: (_reg_e(j, m), 0, 0)),
            pl.BlockSpec((1, 1, OUT_DIM), lambda j, m: (_reg_e(j, m), 0, 0)),
            pl.BlockSpec((1, D_IN, SPIKE_CAP), lambda j, m: (_spk_e(j, m), 0, 0)),
            pl.BlockSpec((1, SPIKE_CAP, OUT_DIM), lambda j, m: (_spk_e(j, m), 0, 0)),
            pl.BlockSpec((1, 1, SPIKE_CAP), lambda j, m: (_spk_e(j, m), 0, 0)),
            pl.BlockSpec((1, 1, OUT_DIM), lambda j, m: (_spk_e(j, m), 0, 0)),
        ],
        out_specs=pl.BlockSpec((BT, OUT_DIM), lambda j, m: (_row_blk(j, m), 0)),
    )
    y = pl.pallas_call(
        _experts_body,
        grid_spec=grid_spec,
        out_shape=jax.ShapeDtypeStruct((A_CAP, OUT_DIM), jnp.float32),
    )(meta, xs,
      W1r.astype(jnp.bfloat16), W2r.astype(jnp.bfloat16),
      b1r[:, None, :], b2r[:, None, :],
      W1s.astype(jnp.bfloat16), W2s.astype(jnp.bfloat16),
      b1s[:, None, :], b2s[:, None, :])

    out = pl.kernel(
        _combine_body,
        out_type=jax.ShapeDtypeStruct((N_TOK, OUT_DIM), jnp.float32),
        mesh=mesh2,
        compiler_params=pltpu.CompilerParams(needs_layout_passes=False),
        scratch_types=[
            pltpu.VMEM((TPW,), jnp.int32),
            pltpu.VMEM((TPW,), jnp.int32),
            pltpu.VMEM((TPW,), jnp.float32),
            pltpu.VMEM((TPW,), jnp.float32),
            pltpu.VMEM((8, OUT_DIM), jnp.float32),
            pltpu.VMEM((8, OUT_DIM), jnp.float32),
            pltpu.VMEM((8, OUT_DIM), jnp.float32),
            pltpu.VMEM((8, OUT_DIM), jnp.float32),
            pltpu.VMEM((8, OUT_DIM), jnp.float32),
            pltpu.VMEM((8, OUT_DIM), jnp.float32),
            pltpu.VMEM((8, OUT_DIM), jnp.float32),
            pltpu.VMEM((8, OUT_DIM), jnp.float32),
            pltpu.VMEM((8, OUT_DIM), jnp.float32),
            pltpu.SemaphoreType.DMA,
            pltpu.SemaphoreType.DMA,
            pltpu.SemaphoreType.DMA,
            pltpu.SemaphoreType.DMA,
            pltpu.SemaphoreType.DMA,
            pltpu.SemaphoreType.DMA,
            pltpu.SemaphoreType.DMA,
            pltpu.SemaphoreType.DMA,
        ],
    )(y, slots, wts)
    return logits, meta, slots, wts, xs, y, out


def kernel(sequence_repr_with_baseline, spike_indicators, Wr, br,
           W1r, b1r, W2r, b2r, W1s, b1s, W2s, b2s):
    return _pipeline(sequence_repr_with_baseline, spike_indicators, Wr, br,
                     W1r, b1r, W2r, b2r, W1s, b1s, W2s, b2s)[-1]


# speedup vs baseline: 1.9808x; 1.0072x over previous
import functools

import jax
import jax.numpy as jnp
from jax import lax
from jax.experimental import pallas as pl
from jax.experimental.pallas import tpu as pltpu
from jax.experimental.pallas import tpu_sc as plsc

N_TOK = 4096
D_IN = 768
NUM_EXPERTS = 8
NUM_SPIKE = 2
TOTAL = NUM_EXPERTS + NUM_SPIKE
EXPERT_DIM = 768
SPIKE_CAP = 1536
OUT_DIM = 768
SPIKE_LEN = 16

BT = 256
NBLK_REG_MAX = 40
NBLK_SPK_MAX = 34
NBLK = 43
A_CAP = NBLK * BT
GRID_E = NBLK - 1

NW = 32
TPW = N_TOK // NW
ROWS_G = A_CAP // NW


def _router_body(x_ref, wr_ref, br_ref, lg_ref):
    res = jnp.dot(x_ref[...], wr_ref[...], preferred_element_type=jnp.float32)
    lg_ref[...] = res + br_ref[0:1, :]


def _phase1_body(lg_hbm, sp_hbm, ek_hbm, wts_hbm, cnts_hbm,
                 lgv, spv, ekv, wkv, cnt):
    c = lax.axis_index("c")
    s = lax.axis_index("s")
    w = s * 2 + c
    base_t = w * TPW
    lane = lax.broadcasted_iota(jnp.int32, (16,), 0)
    zero16i = jnp.zeros((16,), jnp.int32)
    zero16f = jnp.zeros((16,), jnp.float32)

    pltpu.sync_copy(lg_hbm.at[pl.ds(base_t, TPW)], lgv)
    pltpu.sync_copy(sp_hbm.at[pl.ds(base_t, TPW)], spv)

    def grp1_body(g, cntv):
        e0vec = zero16i
        e1vec = zero16i
        p0vec = zero16f
        p1vec = zero16f
        svec = zero16f
        for i in range(16):
            t = g * 16 + i
            lg = lgv[t, :]
            sp = spv[t, :]
            avg = jnp.sum(sp, axis=0) * (1.0 / SPIKE_LEN)
            valid = lane < TOTAL
            spike_m = jnp.logical_and(lane >= NUM_EXPERTS, valid)
            adj = lg + jnp.where(spike_m, avg, 0.0)
            adj = jnp.where(valid, adj, -1e30)
            mx = jnp.max(adj, axis=0)
            p = jnp.where(valid, jnp.exp(adj - mx), 0.0)
            skeys, svals = plsc.sort_key_val(p, lane, descending=True)
            e0 = svals[0]
            e1 = svals[1]
            sel = lane == i
            e0vec = jnp.where(sel, e0, e0vec)
            e1vec = jnp.where(sel, e1, e1vec)
            p0vec = jnp.where(sel, skeys[0], p0vec)
            p1vec = jnp.where(sel, skeys[1], p1vec)
            svec = jnp.where(sel, jnp.sum(p, axis=0), svec)
            cntv = cntv + (lane == e0).astype(jnp.int32)
            cntv = cntv + (lane == e1).astype(jnp.int32)
        pr0 = p0vec / svec
        pr1 = p1vec / svec
        den = pr0 + pr1 + 1e-9
        ekv[0, pl.ds(g * 16, 16)] = e0vec
        ekv[1, pl.ds(g * 16, 16)] = e1vec
        wkv[0, pl.ds(g * 16, 16)] = pr0 / den
        wkv[1, pl.ds(g * 16, 16)] = pr1 / den
        return cntv

    cntv = lax.fori_loop(0, TPW // 16, grp1_body, zero16i)
    cnt[...] = cntv
    pltpu.sync_copy(cnt, cnts_hbm.at[w])
    for k in range(2):
        pltpu.sync_copy(ekv.at[k], ek_hbm.at[pl.ds(k * N_TOK + base_t, TPW)])
        pltpu.sync_copy(wkv.at[k], wts_hbm.at[pl.ds(k * N_TOK + base_t, TPW)])


def _phase2_body(ek_hbm, cnts_hbm, x_hbm, meta_hbm, slots_hbm, xs_hbm,
                 allcnt, ekv, metav, slotsv, xbufs, sem):
    c = lax.axis_index("c")
    s = lax.axis_index("s")
    w = s * 2 + c
    base_t = w * TPW
    lane = lax.broadcasted_iota(jnp.int32, (16,), 0)
    zero16i = jnp.zeros((16,), jnp.int32)

    xstage = pltpu.async_copy(x_hbm.at[pl.ds(base_t, TPW)], xbufs, sem)
    pltpu.sync_copy(cnts_hbm, allcnt)
    tot = zero16i
    mybase = zero16i
    for r in range(NW):
        row = allcnt[r, :]
        tot = tot + row
        mybase = mybase + jnp.where(r < w, row, 0)
    tot = jnp.where(lane < TOTAL, tot, 0)
    padded = ((tot + (BT - 1)) // BT) * BT
    csum = plsc.cumsum(padded)
    off = csum - padded
    csum_blk = csum // BT
    na = csum_blk[TOTAL - 1]

    @pl.when(w == 0)
    def _meta():
        for i in range(8):
            metav[pl.ds(i * 16, 16)] = zero16i
        metav[pl.ds(112, 16)] = jnp.where(lane == 8, na, 0)
        for cch in range(3):
            b = cch * 16 + lane
            acc = zero16i
            for e in range(TOTAL):
                acc = acc + (b >= csum_blk[e]).astype(jnp.int32)
            valid_b = b < na
            pos = jnp.where(valid_b, b, 127)
            val = jnp.where(valid_b, acc, 0)
            plsc.store_scatter(metav, [pos], val)
        pltpu.sync_copy(metav, meta_hbm)

    for k in range(2):
        pltpu.sync_copy(ek_hbm.at[pl.ds(k * N_TOK + base_t, TPW)], ekv.at[k])

    def grp3_body(g, mbv):
        e0vec = ekv[0, pl.ds(g * 16, 16)]
        e1vec = ekv[1, pl.ds(g * 16, 16)]
        slot0 = zero16i
        slot1 = zero16i
        for i in range(16):
            sel = lane == i
            e0 = e0vec[i]
            oh0 = lane == e0
            s0 = jnp.sum(jnp.where(oh0, mbv, 0), axis=0)
            mbv = mbv + oh0.astype(jnp.int32)
            e1 = e1vec[i]
            oh1 = lane == e1
            s1 = jnp.sum(jnp.where(oh1, mbv, 0), axis=0)
            mbv = mbv + oh1.astype(jnp.int32)
            slot0 = jnp.where(sel, s0, slot0)
            slot1 = jnp.where(sel, s1, slot1)
        hi = g // 2
        lo = (g - hi * 2) * 16
        slotsv[hi, pl.ds(lo, 16)] = slot0
        slotsv[4 + hi, pl.ds(lo, 16)] = slot1
        return mbv

    lax.fori_loop(0, TPW // 16, grp3_body, off + mybase)
    for k in range(2):
        for hh in range(4):
            pltpu.sync_copy(
                slotsv.at[k * 4 + hh],
                slots_hbm.at[pl.ds(k * N_TOK + base_t + hh * 32, 32)])
    xstage.wait()
    handles = []
    for k in range(2):
        for hh in range(4):
            handles.append(pltpu.async_copy(
                xbufs.at[pl.ds(hh * 32, 32)],
                xs_hbm.at[slotsv.at[k * 4 + hh]], sem))
    for h in handles:
        h.wait()


def _experts_body(m_ref, xs_ref, w1r_ref, w2r_ref, b1r_ref, b2r_ref,
                  w1s_ref, w2s_ref, b1s_ref, b2s_ref, y_ref):
    j = pl.program_id(0)
    na = m_ref[120]
    e_blk = m_ref[j]

    @pl.when(jnp.logical_and(j < na, e_blk < NUM_EXPERTS))
    def _reg():
        x = xs_ref[...].astype(jnp.bfloat16)
        h = jnp.dot(x, w1r_ref[0], preferred_element_type=jnp.float32)
        h = jnp.maximum(h + b1r_ref[0], 0.0).astype(jnp.bfloat16)
        y_ref[...] = jnp.dot(h, w2r_ref[0],
                             preferred_element_type=jnp.float32) + b2r_ref[0]

    @pl.when(jnp.logical_and(j < na, e_blk >= NUM_EXPERTS))
    def _spk():
        x = xs_ref[...].astype(jnp.bfloat16)
        h = jnp.dot(x, w1s_ref[0], preferred_element_type=jnp.float32)
        h = jnp.maximum(h + b1s_ref[0], 0.0).astype(jnp.bfloat16)
        y_ref[...] = jnp.dot(h, w2s_ref[0],
                             preferred_element_type=jnp.float32) + b2s_ref[0]


def _row_blk(j, m):
    return jnp.where(j < m[120], j, NBLK - 1)


def _reg_e(j, m):
    e = jnp.where(j < m[120], m[j], 0)
    return jnp.where(e < NUM_EXPERTS, e, 0)


def _spk_e(j, m):
    e = jnp.where(j < m[120], m[j], 0)
    return jnp.maximum(e - NUM_EXPERTS, 0)


def _combine_body(y_hbm, slots_hbm, wts_hbm, out_hbm,
                  sl0, sl1, w0v, w1v, *bufs_and_sems):
    ybufs = bufs_and_sems[:8]
    ob = bufs_and_sems[8]
    sems = bufs_and_sems[9:17]
    c = lax.axis_index("c")
    s = lax.axis_index("s")
    w = s * 2 + c
    base = w * TPW
    pltpu.sync_copy(slots_hbm.at[pl.ds(base, TPW)], sl0)
    pltpu.sync_copy(slots_hbm.at[pl.ds(N_TOK + base, TPW)], sl1)
    pltpu.sync_copy(wts_hbm.at[pl.ds(base, TPW)], w0v)
    pltpu.sync_copy(wts_hbm.at[pl.ds(N_TOK + base, TPW)], w1v)
    NCH = TPW // 8
    pairs = tuple((ybufs[2 * p], ybufs[2 * p + 1], sems[2 * p], sems[2 * p + 1])
                  for p in range(4))

    def fire(ci, pb):
        b0, b1, s0_, s1_ = pairs[pb]
        pltpu.async_copy(y_hbm.at[sl0.at[pl.ds(ci * 8, 8)]], b0, s0_)
        pltpu.async_copy(y_hbm.at[sl1.at[pl.ds(ci * 8, 8)]], b1, s1_)

    def drain(pb):
        b0, b1, s0_, s1_ = pairs[pb]
        pltpu.make_async_copy(y_hbm.at[sl0.at[pl.ds(0, 8)]], b0, s0_).wait()
        pltpu.make_async_copy(y_hbm.at[sl1.at[pl.ds(0, 8)]], b1, s1_).wait()

    def compute(ci, pb, half):
        b0, b1, _, _ = pairs[pb]
        w0vec = w0v[pl.ds((ci - half) * 8, 16)]
        w1vec = w1v[pl.ds((ci - half) * 8, 16)]
        for r in range(8):
            w0 = w0vec[half * 8 + r]
            w1 = w1vec[half * 8 + r]
            for sb in range(OUT_DIM // 16):
                sl_ = pl.ds(sb * 16, 16)
                ob[r, sl_] = b0[r, sl_] * w0 + b1[r, sl_] * w1
        pltpu.sync_copy(ob, out_hbm.at[pl.ds(base + ci * 8, 8)])

    for p in range(3):
        fire(p, p)

    def ring_body(g, carry):
        for b in range(4):
            ci = g * 4 + b

            @pl.when(ci + 3 < NCH)
            def _f(ci=ci, b=b):
                fire(ci + 3, (b + 3) % 4)

            drain(b)
            compute(ci, b, b % 2)
        return carry

    lax.fori_loop(0, NCH // 4, ring_body, 0)


def _pipeline(sequence_repr_with_baseline, spike_indicators, Wr, br,
              W1r, b1r, W2r, b2r, W1s, b1s, W2s, b2s):
    x = sequence_repr_with_baseline
    wr_pad = jnp.zeros((D_IN, 16), jnp.float32).at[:, :TOTAL].set(Wr)
    br_pad = jnp.zeros((8, 16), jnp.float32).at[:, :TOTAL].set(
        jnp.broadcast_to(br, (8, TOTAL)))

    logits = pl.pallas_call(
        _router_body,
        grid=(N_TOK // BT,),
        in_specs=[
            pl.BlockSpec((BT, D_IN), lambda i: (i, 0)),
            pl.BlockSpec((D_IN, 16), lambda i: (0, 0)),
            pl.BlockSpec((8, 16), lambda i: (0, 0)),
        ],
        out_specs=pl.BlockSpec((BT, 16), lambda i: (i, 0)),
        out_shape=jax.ShapeDtypeStruct((N_TOK, 16), jnp.float32),
    )(x, wr_pad, br_pad)

    mesh2 = plsc.VectorSubcoreMesh(core_axis_name="c", subcore_axis_name="s")
    ek, wts, cnts = pl.kernel(
        _phase1_body,
        out_type=[
            jax.ShapeDtypeStruct((2 * N_TOK,), jnp.int32),
            jax.ShapeDtypeStruct((2 * N_TOK,), jnp.float32),
            jax.ShapeDtypeStruct((NW, 16), jnp.int32),
        ],
        mesh=mesh2,
        compiler_params=pltpu.CompilerParams(needs_layout_passes=False),
        scratch_types=[
            pltpu.VMEM((TPW, 16), jnp.float32),
            pltpu.VMEM((TPW, 16), jnp.float32),
            pltpu.VMEM((2, TPW), jnp.int32),
            pltpu.VMEM((2, TPW), jnp.float32),
            pltpu.VMEM((16,), jnp.int32),
        ],
    )(logits, spike_indicators)

    meta, slots, xs = pl.kernel(
        _phase2_body,
        out_type=[
            jax.ShapeDtypeStruct((128,), jnp.int32),
            jax.ShapeDtypeStruct((2 * N_TOK,), jnp.int32),
            jax.ShapeDtypeStruct((A_CAP, D_IN), jnp.float32),
        ],
        mesh=mesh2,
        compiler_params=pltpu.CompilerParams(needs_layout_passes=False),
        scratch_types=[
            pltpu.VMEM((NW, 16), jnp.int32),
            pltpu.VMEM((2, TPW), jnp.int32),
            pltpu.VMEM((128,), jnp.int32),
            pltpu.VMEM((8, 32), jnp.int32),
            pltpu.VMEM((TPW, D_IN), jnp.float32),
            pltpu.SemaphoreType.DMA,
        ],
    )(ek, cnts, x)

    grid_spec = pltpu.PrefetchScalarGridSpec(
        num_scalar_prefetch=1,
        grid=(GRID_E,),
        in_specs=[
            pl.BlockSpec((BT, D_IN), lambda j, m: (_row_blk(j, m), 0)),
            pl.BlockSpec((1, D_IN, EXPERT_DIM), lambda j, m: (_reg_e(j, m), 0, 0)),
            pl.BlockSpec((1, EXPERT_DIM, OUT_DIM), lambda j, m: (_reg_e(j, m), 0, 0)),
            pl.BlockSpec((1, 1, EXPERT_DIM), lambda j, m: (_reg_e(j, m), 0, 0)),
            pl.BlockSpec((1, 1, OUT_DIM), lambda j, m: (_reg_e(j, m), 0, 0)),
            pl.BlockSpec((1, D_IN, SPIKE_CAP), lambda j, m: (_spk_e(j, m), 0, 0)),
            pl.BlockSpec((1, SPIKE_CAP, OUT_DIM), lambda j, m: (_spk_e(j, m), 0, 0)),
            pl.BlockSpec((1, 1, SPIKE_CAP), lambda j, m: (_spk_e(j, m), 0, 0)),
            pl.BlockSpec((1, 1, OUT_DIM), lambda j, m: (_spk_e(j, m), 0, 0)),
        ],
        out_specs=pl.BlockSpec((BT, OUT_DIM), lambda j, m: (_row_blk(j, m), 0)),
    )
    y = pl.pallas_call(
        _experts_body,
        grid_spec=grid_spec,
        out_shape=jax.ShapeDtypeStruct((A_CAP, OUT_DIM), jnp.float32),
    )(meta, xs,
      W1r.astype(jnp.bfloat16), W2r.astype(jnp.bfloat16),
      b1r[:, None, :], b2r[:, None, :],
      W1s.astype(jnp.bfloat16), W2s.astype(jnp.bfloat16),
      b1s[:, None, :], b2s[:, None, :])

    out = pl.kernel(
        _combine_body,
        out_type=jax.ShapeDtypeStruct((N_TOK, OUT_DIM), jnp.float32),
        mesh=mesh2,
        compiler_params=pltpu.CompilerParams(needs_layout_passes=False),
        scratch_types=[
            pltpu.VMEM((TPW,), jnp.int32),
            pltpu.VMEM((TPW,), jnp.int32),
            pltpu.VMEM((TPW,), jnp.float32),
            pltpu.VMEM((TPW,), jnp.float32),
            pltpu.VMEM((8, OUT_DIM), jnp.float32),
            pltpu.VMEM((8, OUT_DIM), jnp.float32),
            pltpu.VMEM((8, OUT_DIM), jnp.float32),
            pltpu.VMEM((8, OUT_DIM), jnp.float32),
            pltpu.VMEM((8, OUT_DIM), jnp.float32),
            pltpu.VMEM((8, OUT_DIM), jnp.float32),
            pltpu.VMEM((8, OUT_DIM), jnp.float32),
            pltpu.VMEM((8, OUT_DIM), jnp.float32),
            pltpu.VMEM((8, OUT_DIM), jnp.float32),
            pltpu.SemaphoreType.DMA,
            pltpu.SemaphoreType.DMA,
            pltpu.SemaphoreType.DMA,
            pltpu.SemaphoreType.DMA,
            pltpu.SemaphoreType.DMA,
            pltpu.SemaphoreType.DMA,
            pltpu.SemaphoreType.DMA,
            pltpu.SemaphoreType.DMA,
        ],
    )(y, slots, wts)
    return logits, meta, slots, wts, xs, y, out


def kernel(sequence_repr_with_baseline, spike_indicators, Wr, br,
           W1r, b1r, W2r, b2r, W1s, b1s, W2s, b2s):
    return _pipeline(sequence_repr_with_baseline, spike_indicators, Wr, br,
                     W1r, b1r, W2r, b2r, W1s, b1s, W2s, b2s)[-1]
